# baseline (speedup 1.0000x reference)
import functools

import jax
import jax.numpy as jnp
from jax import lax
from jax.experimental import pallas as pl
from jax.experimental.pallas import tpu as pltpu
from jax.experimental.pallas import tpu_sc as plsc

N_NODES = 100000
MEM_DIM = 128
MSG_DIM = 256
B = 16384
NC = 2
NS = 16
NW = NC * NS
BPW = B // NW

_MESH = dict(core_axis_name="c", subcore_axis_name="s")


def _wid():
  return lax.axis_index("s") * NC + lax.axis_index("c")


_CH = 2048


@functools.partial(
    pl.kernel,
    out_type=jax.ShapeDtypeStruct((B, MEM_DIM), jnp.float32),
    mesh=plsc.VectorSubcoreMesh(**_MESH),
    scratch_types=[
        pltpu.VMEM((BPW,), jnp.int32),
        pltpu.VMEM((BPW, MEM_DIM), jnp.float32),
        pltpu.SemaphoreType.DMA,
    ],
)
def _sc_gather(table_hbm, idx_hbm, out_hbm, idx_v, rows_v, sem):
  base = _wid() * BPW
  pltpu.sync_copy(idx_hbm.at[pl.ds(base, BPW)], idx_v)
  pltpu.async_copy(table_hbm.at[idx_v], rows_v, sem).wait()
  pltpu.sync_copy(rows_v, out_hbm.at[pl.ds(base, BPW)])


@functools.partial(
    pl.kernel,
    out_type=jax.ShapeDtypeStruct((N_NODES,), jnp.int32),
    mesh=plsc.VectorSubcoreMesh(**_MESH),
    scratch_types=[
        pltpu.VMEM((N_NODES,), jnp.int32),
        pltpu.VMEM((B,), jnp.int32),
    ],
    compiler_params=pltpu.CompilerParams(needs_layout_passes=False),
)
def _sc_arb(idx_hbm, pos_hbm, pos_v, ids_v):
  @pl.when(_wid() == 0)
  def _():
    lanes = lax.iota(jnp.int32, 16)
    pltpu.sync_copy(idx_hbm, ids_v)

    @pl.loop(0, B // 16, unroll=8)
    def _v1(v):
      idv = ids_v[pl.ds(v * 16, 16)]
      posv = v * 16 + lanes
      plsc.store_scatter(pos_v, [idv], posv)

    pltpu.sync_copy(pos_v, pos_hbm)


_BM = 1024


_GSTEPS = B // _BM
_CPROWS = N_NODES // _GSTEPS


def _gru_body(tab_any, lu_any, msg_ref, h_ref, wih_ref, whh_ref, bih_ref,
              bhh_ref, out_ref, tabout_any, luout_any, sem, sem_lu):
  i = pl.program_id(0)
  pltpu.async_copy(tab_any.at[pl.ds(i * _CPROWS, _CPROWS)],
                   tabout_any.at[pl.ds(i * _CPROWS, _CPROWS)], sem)

  @pl.when(i == 0)
  def _():
    pltpu.async_copy(lu_any, luout_any, sem_lu)

  msg = msg_ref[...]
  h = h_ref[...]
  dn = (((1,), (1,)), ((), ()))
  gi = lax.dot_general(msg, wih_ref[...], dn,
                       preferred_element_type=jnp.float32) + bih_ref[...]
  gh = lax.dot_general(h, whh_ref[...], dn,
                       preferred_element_type=jnp.float32) + bhh_ref[...]
  r = jax.nn.sigmoid(gi[:, :MEM_DIM] + gh[:, :MEM_DIM])
  z = jax.nn.sigmoid(gi[:, MEM_DIM:2 * MEM_DIM] + gh[:, MEM_DIM:2 * MEM_DIM])
  n = jnp.tanh(gi[:, 2 * MEM_DIM:] + r * gh[:, 2 * MEM_DIM:])
  out_ref[...] = (1.0 - z) * n + z * h

  @pl.when(i == _GSTEPS - 1)
  def _():
    for j in range(_GSTEPS):
      pltpu.make_async_copy(
          tab_any.at[pl.ds(j * _CPROWS, _CPROWS)],
          tabout_any.at[pl.ds(j * _CPROWS, _CPROWS)], sem).wait()
    pltpu.make_async_copy(lu_any, luout_any, sem_lu).wait()


_tc_gru = pl.pallas_call(
    _gru_body,
    out_shape=(
        jax.ShapeDtypeStruct((B, MEM_DIM), jnp.float32),
        jax.ShapeDtypeStruct((N_NODES, MEM_DIM), jnp.float32),
        jax.ShapeDtypeStruct((N_NODES,), jnp.float32),
    ),
    grid=(_GSTEPS,),
    in_specs=[
        pl.BlockSpec(memory_space=pl.ANY),
        pl.BlockSpec(memory_space=pl.ANY),
        pl.BlockSpec((_BM, MSG_DIM), lambda i: (i, 0)),
        pl.BlockSpec((_BM, MEM_DIM), lambda i: (i, 0)),
        pl.BlockSpec((3 * MEM_DIM, MSG_DIM), lambda i: (0, 0)),
        pl.BlockSpec((3 * MEM_DIM, MEM_DIM), lambda i: (0, 0)),
        pl.BlockSpec((1, 3 * MEM_DIM), lambda i: (0, 0)),
        pl.BlockSpec((1, 3 * MEM_DIM), lambda i: (0, 0)),
    ],
    out_specs=(
        pl.BlockSpec((_BM, MEM_DIM), lambda i: (i, 0)),
        pl.BlockSpec(memory_space=pl.ANY),
        pl.BlockSpec(memory_space=pl.ANY),
    ),
    scratch_shapes=[pltpu.SemaphoreType.DMA, pltpu.SemaphoreType.DMA],
    compiler_params=pltpu.CompilerParams(
        dimension_semantics=("arbitrary",)),
)


_NCHUNK = 4
_CROWS = BPW // _NCHUNK


@functools.partial(
    pl.kernel,
    out_type=(),
    mesh=plsc.VectorSubcoreMesh(**_MESH),
    scratch_types=[
        pltpu.VMEM((_NCHUNK, _CROWS), jnp.int32),
        pltpu.VMEM((_NCHUNK, _CROWS), jnp.int32),
        pltpu.VMEM((BPW, MEM_DIM), jnp.float32),
        pltpu.VMEM((BPW,), jnp.float32),
        [pltpu.SemaphoreType.DMA] * _NCHUNK,
        [pltpu.SemaphoreType.DMA] * _NCHUNK,
        pltpu.SemaphoreType.DMA,
        pltpu.SemaphoreType.DMA,
    ],
)
def _sc_scatter(tab_ref, lu_ref, idx_hbm, pos_hbm, hnew_hbm, ts_hbm,
                idx_v, w_v, rows_v, ts_v, sems_w, sems_r, sem_ts, sem_s):
  wid = _wid()
  pltpu.sync_copy(idx_hbm.at[wid], idx_v)
  w_gathers = [
      pltpu.async_copy(pos_hbm.at[idx_v.at[c]], w_v.at[c], sems_w[c])
      for c in range(_NCHUNK)
  ]
  gathers = []
  for c in range(_NCHUNK):
    w_gathers[c].wait()
    gathers.append(pltpu.async_copy(
        hnew_hbm.at[w_v.at[c]], rows_v.at[pl.ds(c * _CROWS, _CROWS)],
        sems_r[c]))
  ts_gathers = [
      pltpu.async_copy(ts_hbm.at[w_v.at[c]],
                       ts_v.at[pl.ds(c * _CROWS, _CROWS)], sems_w[c])
      for c in range(_NCHUNK)
  ]
  scatters = []
  for c in range(_NCHUNK):
    gathers[c].wait()
    scatters.append(pltpu.async_copy(
        rows_v.at[pl.ds(c * _CROWS, _CROWS)], tab_ref.at[idx_v.at[c]],
        sems_r[c]))
  for c in range(_NCHUNK):
    ts_gathers[c].wait()
  ts_scatters = [
      pltpu.async_copy(ts_v.at[pl.ds(c * _CROWS, _CROWS)],
                       lu_ref.at[idx_v.at[c]], sem_s)
      for c in range(_NCHUNK)
  ]
  for c in range(_NCHUNK):
    scatters[c].wait()
    ts_scatters[c].wait()


def kernel(memory_table, last_update, unique_node_ids, unique_messages,
           timestamps, W_ih, W_hh, b_ih, b_hh):
  h = _sc_gather(memory_table, unique_node_ids)
  pos = _sc_arb(unique_node_ids)
  h_new, tab_out, lu_out = _tc_gru(memory_table, last_update,
                                   unique_messages, h, W_ih, W_hh,
                                   b_ih.reshape(1, -1), b_hh.reshape(1, -1))
  tab_ref = jax.new_ref(tab_out)
  lu_ref = jax.new_ref(lu_out)
  ids3 = unique_node_ids.reshape(NW, _NCHUNK, _CROWS)
  _sc_scatter(tab_ref, lu_ref, ids3, pos, h_new, timestamps)
  return tab_ref[...], lu_ref[...]


# speedup vs baseline: 14.1405x; 14.1405x over previous
import functools

import jax
import jax.numpy as jnp
from jax import lax
from jax.experimental import pallas as pl
from jax.experimental.pallas import tpu as pltpu
from jax.experimental.pallas import tpu_sc as plsc

N_NODES = 100000
MEM_DIM = 128
MSG_DIM = 256
B = 16384
NC = 2
NS = 16
NW = NC * NS
BPW = B // NW

_MESH = dict(core_axis_name="c", subcore_axis_name="s")


def _wid():
  return lax.axis_index("s") * NC + lax.axis_index("c")


_CH = 2048


@functools.partial(
    pl.kernel,
    out_type=jax.ShapeDtypeStruct((B, MEM_DIM), jnp.float32),
    mesh=plsc.VectorSubcoreMesh(**_MESH),
    scratch_types=[
        pltpu.VMEM((BPW,), jnp.int32),
        pltpu.VMEM((BPW, MEM_DIM), jnp.float32),
        pltpu.SemaphoreType.DMA,
    ],
)
def _sc_gather(table_hbm, idx_hbm, out_hbm, idx_v, rows_v, sem):
  base = _wid() * BPW
  pltpu.sync_copy(idx_hbm.at[pl.ds(base, BPW)], idx_v)
  pltpu.async_copy(table_hbm.at[idx_v], rows_v, sem).wait()
  pltpu.sync_copy(rows_v, out_hbm.at[pl.ds(base, BPW)])


@functools.partial(
    pl.kernel,
    out_type=jax.ShapeDtypeStruct((N_NODES,), jnp.int32),
    mesh=plsc.VectorSubcoreMesh(**_MESH),
    scratch_types=[
        pltpu.VMEM((N_NODES,), jnp.int32),
        pltpu.VMEM((B,), jnp.int32),
    ],
    compiler_params=pltpu.CompilerParams(needs_layout_passes=False),
)
def _sc_arb(idx_hbm, pos_hbm, pos_v, ids_v):
  @pl.when(_wid() == 0)
  def _():
    lanes = lax.iota(jnp.int32, 16)
    pltpu.sync_copy(idx_hbm, ids_v)

    @pl.loop(0, B // 16, unroll=8)
    def _v1(v):
      idv = ids_v[pl.ds(v * 16, 16)]
      posv = v * 16 + lanes
      plsc.store_scatter(pos_v, [idv], posv)

    pltpu.sync_copy(pos_v, pos_hbm)


_BM = 1024


def _gru_body(msg_ref, h_ref, wih_ref, whh_ref, bih_ref, bhh_ref, out_ref):
  msg = msg_ref[...]
  h = h_ref[...]
  dn = (((1,), (1,)), ((), ()))
  gi = lax.dot_general(msg, wih_ref[...], dn,
                       preferred_element_type=jnp.float32) + bih_ref[...]
  gh = lax.dot_general(h, whh_ref[...], dn,
                       preferred_element_type=jnp.float32) + bhh_ref[...]
  r = jax.nn.sigmoid(gi[:, :MEM_DIM] + gh[:, :MEM_DIM])
  z = jax.nn.sigmoid(gi[:, MEM_DIM:2 * MEM_DIM] + gh[:, MEM_DIM:2 * MEM_DIM])
  n = jnp.tanh(gi[:, 2 * MEM_DIM:] + r * gh[:, 2 * MEM_DIM:])
  out_ref[...] = (1.0 - z) * n + z * h


_tc_gru = pl.pallas_call(
    _gru_body,
    out_shape=jax.ShapeDtypeStruct((B, MEM_DIM), jnp.float32),
    grid=(B // _BM,),
    in_specs=[
        pl.BlockSpec((_BM, MSG_DIM), lambda i: (i, 0)),
        pl.BlockSpec((_BM, MEM_DIM), lambda i: (i, 0)),
        pl.BlockSpec((3 * MEM_DIM, MSG_DIM), lambda i: (0, 0)),
        pl.BlockSpec((3 * MEM_DIM, MEM_DIM), lambda i: (0, 0)),
        pl.BlockSpec((1, 3 * MEM_DIM), lambda i: (0, 0)),
        pl.BlockSpec((1, 3 * MEM_DIM), lambda i: (0, 0)),
    ],
    out_specs=pl.BlockSpec((_BM, MEM_DIM), lambda i: (i, 0)),
    compiler_params=pltpu.CompilerParams(
        dimension_semantics=("parallel",)),
)


_CPB = 4000


def _copy_body(in_ref, out_ref):
  out_ref[...] = in_ref[...]


_tc_copy = pl.pallas_call(
    _copy_body,
    out_shape=jax.ShapeDtypeStruct((N_NODES, MEM_DIM), jnp.float32),
    grid=(N_NODES // _CPB,),
    in_specs=[pl.BlockSpec((_CPB, MEM_DIM), lambda i: (i, 0))],
    out_specs=pl.BlockSpec((_CPB, MEM_DIM), lambda i: (i, 0)),
    compiler_params=pltpu.CompilerParams(
        dimension_semantics=("parallel",)),
)


_NCHUNK = 4
_CROWS = BPW // _NCHUNK


@functools.partial(
    pl.kernel,
    out_type=(),
    mesh=plsc.VectorSubcoreMesh(**_MESH),
    scratch_types=[
        pltpu.VMEM((_NCHUNK, _CROWS), jnp.int32),
        pltpu.VMEM((_NCHUNK, _CROWS), jnp.int32),
        pltpu.VMEM((BPW, MEM_DIM), jnp.float32),
        pltpu.VMEM((BPW,), jnp.float32),
        [pltpu.SemaphoreType.DMA] * _NCHUNK,
        [pltpu.SemaphoreType.DMA] * _NCHUNK,
        pltpu.SemaphoreType.DMA,
        pltpu.SemaphoreType.DMA,
    ],
)
def _sc_scatter(tab_ref, lu_ref, idx_hbm, pos_hbm, hnew_hbm, ts_hbm,
                idx_v, w_v, rows_v, ts_v, sems_w, sems_r, sem_ts, sem_s):
  wid = _wid()
  pltpu.sync_copy(idx_hbm.at[wid], idx_v)
  w_gathers = [
      pltpu.async_copy(pos_hbm.at[idx_v.at[c]], w_v.at[c], sems_w[c])
      for c in range(_NCHUNK)
  ]
  gathers = []
  for c in range(_NCHUNK):
    w_gathers[c].wait()
    gathers.append(pltpu.async_copy(
        hnew_hbm.at[w_v.at[c]], rows_v.at[pl.ds(c * _CROWS, _CROWS)],
        sems_r[c]))
  ts_gathers = [
      pltpu.async_copy(ts_hbm.at[w_v.at[c]],
                       ts_v.at[pl.ds(c * _CROWS, _CROWS)], sems_w[c])
      for c in range(_NCHUNK)
  ]
  scatters = []
  for c in range(_NCHUNK):
    gathers[c].wait()
    scatters.append(pltpu.async_copy(
        rows_v.at[pl.ds(c * _CROWS, _CROWS)], tab_ref.at[idx_v.at[c]],
        sems_r[c]))
  for c in range(_NCHUNK):
    ts_gathers[c].wait()
  ts_scatters = [
      pltpu.async_copy(ts_v.at[pl.ds(c * _CROWS, _CROWS)],
                       lu_ref.at[idx_v.at[c]], sem_s)
      for c in range(_NCHUNK)
  ]
  for c in range(_NCHUNK):
    scatters[c].wait()
    ts_scatters[c].wait()


def kernel(memory_table, last_update, unique_node_ids, unique_messages,
           timestamps, W_ih, W_hh, b_ih, b_hh):
  h = _sc_gather(memory_table, unique_node_ids)
  pos = _sc_arb(unique_node_ids)
  tab_out = _tc_copy(memory_table)
  h_new = _tc_gru(unique_messages, h, W_ih, W_hh,
                  b_ih.reshape(1, -1), b_hh.reshape(1, -1))
  tab_ref = jax.new_ref(tab_out)
  lu_ref = jax.new_ref(last_update)
  ids3 = unique_node_ids.reshape(NW, _NCHUNK, _CROWS)
  _sc_scatter(tab_ref, lu_ref, ids3, pos, h_new, timestamps)
  return tab_ref[...], lu_ref[...]


# speedup vs baseline: 15.1534x; 1.0716x over previous
import functools

import jax
import jax.numpy as jnp
from jax import lax
from jax.experimental import pallas as pl
from jax.experimental.pallas import tpu as pltpu
from jax.experimental.pallas import tpu_sc as plsc

N_NODES = 100000
MEM_DIM = 128
MSG_DIM = 256
B = 16384
NC = 2
NS = 16
NW = NC * NS
BPW = B // NW

_MESH = dict(core_axis_name="c", subcore_axis_name="s")


def _wid():
  return lax.axis_index("s") * NC + lax.axis_index("c")


_CH = 2048


@functools.partial(
    pl.kernel,
    out_type=jax.ShapeDtypeStruct((B, MEM_DIM), jnp.float32),
    mesh=plsc.VectorSubcoreMesh(**_MESH),
    scratch_types=[
        pltpu.VMEM((BPW,), jnp.int32),
        pltpu.VMEM((BPW, MEM_DIM), jnp.float32),
        pltpu.SemaphoreType.DMA,
    ],
)
def _sc_gather(table_hbm, idx_hbm, out_hbm, idx_v, rows_v, sem):
  base = _wid() * BPW
  pltpu.sync_copy(idx_hbm.at[pl.ds(base, BPW)], idx_v)
  pltpu.async_copy(table_hbm.at[idx_v], rows_v, sem).wait()
  pltpu.sync_copy(rows_v, out_hbm.at[pl.ds(base, BPW)])


@functools.partial(
    pl.kernel,
    out_type=jax.ShapeDtypeStruct((N_NODES,), jnp.int32),
    mesh=plsc.VectorSubcoreMesh(**_MESH),
    scratch_types=[
        pltpu.VMEM((N_NODES,), jnp.int32),
        pltpu.VMEM((B,), jnp.int32),
    ],
    compiler_params=pltpu.CompilerParams(needs_layout_passes=False),
)
def _sc_arb(idx_hbm, pos_hbm, pos_v, ids_v):
  @pl.when(_wid() == 0)
  def _():
    lanes = lax.iota(jnp.int32, 16)
    pltpu.sync_copy(idx_hbm, ids_v)

    @pl.loop(0, B // 16, unroll=8)
    def _v1(v):
      idv = ids_v[pl.ds(v * 16, 16)]
      posv = v * 16 + lanes
      plsc.store_scatter(pos_v, [idv], posv)

    pltpu.sync_copy(pos_v, pos_hbm)


_BM = 2048


def _gru_body(msg_ref, h_ref, wih_ref, whh_ref, bih_ref, bhh_ref, out_ref):
  msg = msg_ref[...]
  h = h_ref[...]
  dn = (((1,), (1,)), ((), ()))
  gi = lax.dot_general(msg, wih_ref[...], dn,
                       preferred_element_type=jnp.float32) + bih_ref[...]
  gh = lax.dot_general(h, whh_ref[...], dn,
                       preferred_element_type=jnp.float32) + bhh_ref[...]
  r = jax.nn.sigmoid(gi[:, :MEM_DIM] + gh[:, :MEM_DIM])
  z = jax.nn.sigmoid(gi[:, MEM_DIM:2 * MEM_DIM] + gh[:, MEM_DIM:2 * MEM_DIM])
  n = jnp.tanh(gi[:, 2 * MEM_DIM:] + r * gh[:, 2 * MEM_DIM:])
  out_ref[...] = (1.0 - z) * n + z * h


_tc_gru = pl.pallas_call(
    _gru_body,
    out_shape=jax.ShapeDtypeStruct((B, MEM_DIM), jnp.float32),
    grid=(B // _BM,),
    in_specs=[
        pl.BlockSpec((_BM, MSG_DIM), lambda i: (i, 0)),
        pl.BlockSpec((_BM, MEM_DIM), lambda i: (i, 0)),
        pl.BlockSpec((3 * MEM_DIM, MSG_DIM), lambda i: (0, 0)),
        pl.BlockSpec((3 * MEM_DIM, MEM_DIM), lambda i: (0, 0)),
        pl.BlockSpec((1, 3 * MEM_DIM), lambda i: (0, 0)),
        pl.BlockSpec((1, 3 * MEM_DIM), lambda i: (0, 0)),
    ],
    out_specs=pl.BlockSpec((_BM, MEM_DIM), lambda i: (i, 0)),
    compiler_params=pltpu.CompilerParams(
        dimension_semantics=("parallel",)),
)


_CPB = 10000


def _copy_body(in_ref, out_ref):
  out_ref[...] = in_ref[...]


_tc_copy = pl.pallas_call(
    _copy_body,
    out_shape=jax.ShapeDtypeStruct((N_NODES, MEM_DIM), jnp.float32),
    grid=(N_NODES // _CPB,),
    in_specs=[pl.BlockSpec((_CPB, MEM_DIM), lambda i: (i, 0))],
    out_specs=pl.BlockSpec((_CPB, MEM_DIM), lambda i: (i, 0)),
    compiler_params=pltpu.CompilerParams(
        dimension_semantics=("parallel",)),
)


_NCHUNK = 4
_CROWS = BPW // _NCHUNK


@functools.partial(
    pl.kernel,
    out_type=(),
    mesh=plsc.VectorSubcoreMesh(**_MESH),
    scratch_types=[
        pltpu.VMEM((_NCHUNK, _CROWS), jnp.int32),
        pltpu.VMEM((_NCHUNK, _CROWS), jnp.int32),
        pltpu.VMEM((BPW, MEM_DIM), jnp.float32),
        pltpu.VMEM((BPW,), jnp.float32),
        [pltpu.SemaphoreType.DMA] * _NCHUNK,
        [pltpu.SemaphoreType.DMA] * _NCHUNK,
        pltpu.SemaphoreType.DMA,
        pltpu.SemaphoreType.DMA,
    ],
)
def _sc_scatter(tab_ref, lu_ref, idx_hbm, pos_hbm, hnew_hbm, ts_hbm,
                idx_v, w_v, rows_v, ts_v, sems_w, sems_r, sem_ts, sem_s):
  wid = _wid()
  pltpu.sync_copy(idx_hbm.at[wid], idx_v)
  w_gathers = [
      pltpu.async_copy(pos_hbm.at[idx_v.at[c]], w_v.at[c], sems_w[c])
      for c in range(_NCHUNK)
  ]
  gathers = []
  for c in range(_NCHUNK):
    w_gathers[c].wait()
    gathers.append(pltpu.async_copy(
        hnew_hbm.at[w_v.at[c]], rows_v.at[pl.ds(c * _CROWS, _CROWS)],
        sems_r[c]))
  ts_gathers = [
      pltpu.async_copy(ts_hbm.at[w_v.at[c]],
                       ts_v.at[pl.ds(c * _CROWS, _CROWS)], sems_w[c])
      for c in range(_NCHUNK)
  ]
  scatters = []
  for c in range(_NCHUNK):
    gathers[c].wait()
    scatters.append(pltpu.async_copy(
        rows_v.at[pl.ds(c * _CROWS, _CROWS)], tab_ref.at[idx_v.at[c]],
        sems_r[c]))
  for c in range(_NCHUNK):
    ts_gathers[c].wait()
  ts_scatters = [
      pltpu.async_copy(ts_v.at[pl.ds(c * _CROWS, _CROWS)],
                       lu_ref.at[idx_v.at[c]], sem_s)
      for c in range(_NCHUNK)
  ]
  for c in range(_NCHUNK):
    scatters[c].wait()
    ts_scatters[c].wait()


def kernel(memory_table, last_update, unique_node_ids, unique_messages,
           timestamps, W_ih, W_hh, b_ih, b_hh):
  h = _sc_gather(memory_table, unique_node_ids)
  pos = _sc_arb(unique_node_ids)
  tab_out = _tc_copy(memory_table)
  h_new = _tc_gru(unique_messages, h, W_ih, W_hh,
                  b_ih.reshape(1, -1), b_hh.reshape(1, -1))
  tab_ref = jax.new_ref(tab_out)
  lu_ref = jax.new_ref(last_update)
  ids3 = unique_node_ids.reshape(NW, _NCHUNK, _CROWS)
  _sc_scatter(tab_ref, lu_ref, ids3, pos, h_new, timestamps)
  return tab_ref[...], lu_ref[...]
